# baseline (speedup 1.0000x reference)
import functools

import jax
import jax.numpy as jnp
from jax import lax
from jax.experimental import pallas as pl
from jax.experimental.pallas import tpu as pltpu
from jax.experimental.pallas import tpu_sc as plsc


def _make_edge_kernel(NP, WG, EP, CH):
  NSUB = CH // 128
  EPT = EP // 32
  NCHUNK = EPT // CH
  NPAIR = NCHUNK // 2
  RPT = NP // 16
  assert NCHUNK % 2 == 0

  mesh = plsc.VectorSubcoreMesh(core_axis_name="c", subcore_axis_name="s")

  @functools.partial(
      pl.kernel,
      mesh=mesh,
      compiler_params=pltpu.CompilerParams(
          needs_layout_passes=False, use_tc_tiling_on_sc=False),
      out_type=jax.ShapeDtypeStruct((2 * NP, WG), jnp.float32),
      scratch_types=[
          pltpu.VMEM((NP,), jnp.float32),
          pltpu.VMEM((NP,), jnp.float32),
          pltpu.VMEM((CH,), jnp.int32),
          pltpu.VMEM((NSUB, 128), jnp.int32),
          pltpu.VMEM((CH,), jnp.int32),
          pltpu.VMEM((NSUB, 128), jnp.int32),
          pltpu.VMEM((CH,), jnp.float32),
          pltpu.VMEM((CH,), jnp.float32),
          pltpu.VMEM((CH, WG), jnp.float32),
          pltpu.VMEM((CH, WG), jnp.float32),
          pltpu.VMEM_SHARED((NP, WG), jnp.float32),
          pltpu.VMEM_SHARED((NP, WG), jnp.float32),
          pltpu.SemaphoreType.DMA,
          pltpu.SemaphoreType.DMA,
          pltpu.SemaphoreType.DMA,
          pltpu.SemaphoreType.DMA,
      ],
  )
  def edge_kernel(gext, asrc, adst, srcp, dstp2, zinit, out,
                  av_s, av_d, sbufA, dbuf2A, sbufB, dbuf2B, wbufA, wbufB,
                  rows0, rows1, accS, gextS, sem_i, sem_g0, sem_g1, sem_s1):
    cid = lax.axis_index("c")
    sid = lax.axis_index("s")
    wid = cid * 16 + sid
    iota16 = lax.iota(jnp.int32, 16)
    zero16 = jnp.zeros((16,), jnp.int32)
    tile_base = wid * EPT

    def load_idx(base, sb, db2):
      row0 = pl.multiple_of(base // 128, 8)
      c1 = pltpu.async_copy(srcp.at[pl.ds(base, CH)], sb, sem_i)
      c2 = pltpu.async_copy(dstp2.at[pl.ds(row0, NSUB)], db2, sem_i)
      c1.wait()
      c2.wait()

    def fire_gathers(sb, rws, sem):
      return [
          pltpu.async_copy(gextS.at[sb.at[pl.ds(j * 128, 128)]],
                           rws.at[pl.ds(j * 128, 128)], sem)
          for j in range(NSUB)
      ]

    def drain_gathers(sem, sb, rws):
      for j in range(NSUB):
        pltpu.make_async_copy(gextS.at[sb.at[pl.ds(j * 128, 128)]],
                              rws.at[pl.ds(j * 128, 128)], sem).wait()

    def fire_scatters(rws, db2, sem):
      return [
          pltpu.async_copy(rws.at[pl.ds(j * 128, 128)],
                           accS.at[db2.at[j]], sem, add=True)
          for j in range(NSUB)
      ]

    def drain_scatters(sem, rws, db2):
      for j in range(NSUB):
        pltpu.make_async_copy(rws.at[pl.ds(j * 128, 128)],
                              accS.at[db2.at[j]], sem).wait()

    def compute_w(sb, db2, wb):
      for j in range(NSUB):
        def grp(gg, carry2, j=j):
          g = j * 8 + gg
          s16 = sb[pl.ds(g * 16, 16)]
          d16 = db2[j, pl.ds(gg * 16, 16)]
          a_s = plsc.load_gather(av_s, [s16])
          a_d = plsc.load_gather(av_d, [d16])
          e = a_s + a_d
          e = jnp.where(e >= 0.0, e, 0.2 * e)
          wb[pl.ds(g * 16, 16)] = jnp.exp(e)
          return carry2
        lax.fori_loop(0, 8, grp, 0)

    def scale_rows(rws, wb):
      def grp(g, carry2):
        w = wb[pl.ds(g * 16, 16)]
        ridx = g * 16 + iota16
        for c in range(WG - 1):
          cidx = jnp.full((16,), c, jnp.int32)
          v = plsc.load_gather(rws, [ridx, cidx])
          plsc.store_scatter(rws, [ridx, cidx], v * w)
        plsc.store_scatter(rws, [ridx, jnp.full((16,), WG - 1, jnp.int32)], w)
        return carry2
      lax.fori_loop(0, CH // 16, grp, 0)

    pltpu.sync_copy(asrc, av_s)
    pltpu.sync_copy(adst, av_d)

    pltpu.sync_copy(zinit.at[pl.ds(sid * RPT, RPT)],
                    accS.at[pl.ds(sid * RPT, RPT)])
    pltpu.sync_copy(gext.at[pl.ds(sid * RPT, RPT)],
                    gextS.at[pl.ds(sid * RPT, RPT)])
    plsc.subcore_barrier()

    load_idx(tile_base, sbufA, dbuf2A)
    fire_gathers(sbufA, rows0, sem_g0)

    def pair_body(p, carry):
      baseA = pl.multiple_of(tile_base + (2 * p) * CH, CH)
      baseB = pl.multiple_of(baseA + CH, CH)

      @pl.when(p > 0)
      def _():
        drain_scatters(sem_s1, rows1, dbuf2B)

      load_idx(baseB, sbufB, dbuf2B)
      compute_w(sbufA, dbuf2A, wbufA)
      drain_gathers(sem_g0, sbufA, rows0)
      hg1 = fire_gathers(sbufB, rows1, sem_g1)
      scale_rows(rows0, wbufA)
      hs0 = fire_scatters(rows0, dbuf2A, sem_g0)
      compute_w(sbufB, dbuf2B, wbufB)
      for h in hg1:
        h.wait()
      for h in hs0:
        h.wait()
      nxt = jnp.minimum(2 * p + 2, NCHUNK - 1)
      baseN = pl.multiple_of(tile_base + nxt * CH, CH)
      load_idx(baseN, sbufA, dbuf2A)

      @pl.when(p < NPAIR - 1)
      def _():
        fire_gathers(sbufA, rows0, sem_g0)

      scale_rows(rows1, wbufB)
      fire_scatters(rows1, dbuf2B, sem_s1)
      return carry

    lax.fori_loop(0, NPAIR, pair_body, 0)
    drain_scatters(sem_s1, rows1, dbuf2B)

    plsc.subcore_barrier()
    pltpu.sync_copy(accS.at[pl.ds(sid * RPT, RPT)],
                    out.at[pl.ds(cid * NP + sid * RPT, RPT)])

  return edge_kernel


def _tc_layer1(x, W1, a1s, a1d, NP):
  N, _ = x.shape
  H = W1.shape[1]
  zrows = NP - N

  def body(x_ref, w_ref, s_ref, d_ref, g_ref, as_ref, ad_ref):
    h = jnp.dot(x_ref[...], w_ref[...], preferred_element_type=jnp.float32)
    g = jnp.concatenate([h, jnp.ones((N, 1), jnp.float32)], axis=1)
    g_ref[...] = jnp.concatenate(
        [g, jnp.zeros((zrows, H + 1), jnp.float32)], axis=0)
    zp = jnp.zeros((zrows, 1), jnp.float32)
    asv = jnp.dot(h, s_ref[...], preferred_element_type=jnp.float32)
    adv = jnp.dot(h, d_ref[...], preferred_element_type=jnp.float32)
    as_ref[...] = jnp.concatenate([asv, zp], axis=0)[:, 0]
    ad_ref[...] = jnp.concatenate([adv, zp], axis=0)[:, 0]

  return pl.pallas_call(
      body,
      out_shape=[
          jax.ShapeDtypeStruct((NP, H + 1), jnp.float32),
          jax.ShapeDtypeStruct((NP,), jnp.float32),
          jax.ShapeDtypeStruct((NP,), jnp.float32),
      ],
  )(x, W1, a1s.reshape(H, 1), a1d.reshape(H, 1))


def _tc_layer2(acc1, b1, W2, a2s, a2d, NP):
  H1 = W2.shape[0]
  H2 = W2.shape[1]

  def body(a_ref, b_ref, w_ref, s_ref, d_ref, g_ref, as_ref, ad_ref):
    a = a_ref[:NP] + a_ref[NP:]
    z = a[:, :H1] / (a[:, H1:] + 1e-16) + b_ref[...]
    z = jnp.maximum(z, 0.0)
    h2 = jnp.dot(z, w_ref[...], preferred_element_type=jnp.float32)
    g_ref[...] = jnp.concatenate(
        [h2, jnp.ones((NP, 1), jnp.float32)], axis=1)
    as_ref[...] = jnp.dot(
        h2, s_ref[...], preferred_element_type=jnp.float32)[:, 0]
    ad_ref[...] = jnp.dot(
        h2, d_ref[...], preferred_element_type=jnp.float32)[:, 0]

  return pl.pallas_call(
      body,
      out_shape=[
          jax.ShapeDtypeStruct((NP, H2 + 1), jnp.float32),
          jax.ShapeDtypeStruct((NP,), jnp.float32),
          jax.ShapeDtypeStruct((NP,), jnp.float32),
      ],
  )(acc1, b1.reshape(1, H1), W2, a2s.reshape(H2, 1), a2d.reshape(H2, 1))


def _tc_final(acc2, b2, Wf, bf, NP, N):
  H2 = Wf.shape[0]
  C = Wf.shape[1]

  def body(a_ref, b_ref, w_ref, bf_ref, o_ref):
    a = a_ref[:N] + a_ref[NP:NP + N]
    z = a[:, :H2] / (a[:, H2:] + 1e-16) + b_ref[...]
    o_ref[...] = jnp.dot(
        z, w_ref[...], preferred_element_type=jnp.float32) + bf_ref[...]

  return pl.pallas_call(
      body,
      out_shape=jax.ShapeDtypeStruct((N, C), jnp.float32),
  )(acc2, b2.reshape(1, H2), Wf, bf.reshape(1, C))


def kernel(x, edge_index, W1, a1_src, a1_dst, b1, W2, a2_src, a2_dst, b2,
           Wf, bf):
  N = x.shape[0]
  E = edge_index.shape[1]
  H1 = W1.shape[1]
  H2 = W2.shape[1]

  CH = 1024
  NP = ((N + 128) + 2047) // 2048 * 2048
  EP = (E + 32 * CH - 1) // (32 * CH) * (32 * CH)

  pad = EP - E
  src = edge_index[0]
  dst = edge_index[1]
  if pad:
    pidx = jnp.arange(pad, dtype=jnp.int32)
    src = jnp.concatenate([src, pidx % N])
    dst = jnp.concatenate([dst, N + (pidx % 128)])
  dst2 = dst.reshape(EP // 128, 128)

  gext1, as1, ad1 = _tc_layer1(x, W1, a1_src, a1_dst, NP)
  ek1 = _make_edge_kernel(NP, H1 + 1, EP, CH)
  acc1 = ek1(gext1, as1, ad1, src, dst2,
             jnp.zeros((NP, H1 + 1), jnp.float32))

  gext2, as2, ad2 = _tc_layer2(acc1, b1, W2, a2_src, a2_dst, NP)
  ek2 = _make_edge_kernel(NP, H2 + 1, EP, CH)
  acc2 = ek2(gext2, as2, ad2, src, dst2,
             jnp.zeros((NP, H2 + 1), jnp.float32))

  return _tc_final(acc2, b2, Wf, bf, NP, N)


# speedup vs baseline: 1.0065x; 1.0065x over previous
import functools

import jax
import jax.numpy as jnp
from jax import lax
from jax.experimental import pallas as pl
from jax.experimental.pallas import tpu as pltpu
from jax.experimental.pallas import tpu_sc as plsc


def _make_edge_kernel(NP, WG, EP, CH):
  NSUB = CH // 128
  EPT = EP // 32
  NCHUNK = EPT // CH
  NPAIR = NCHUNK // 2
  RPT = NP // 16
  assert NCHUNK % 2 == 0

  mesh = plsc.VectorSubcoreMesh(core_axis_name="c", subcore_axis_name="s")

  @functools.partial(
      pl.kernel,
      mesh=mesh,
      compiler_params=pltpu.CompilerParams(
          needs_layout_passes=False, use_tc_tiling_on_sc=False),
      out_type=jax.ShapeDtypeStruct((2 * NP, WG), jnp.float32),
      scratch_types=[
          pltpu.VMEM((NP,), jnp.float32),
          pltpu.VMEM((NP,), jnp.float32),
          pltpu.VMEM((CH,), jnp.int32),
          pltpu.VMEM((NSUB, 128), jnp.int32),
          pltpu.VMEM((CH,), jnp.int32),
          pltpu.VMEM((NSUB, 128), jnp.int32),
          pltpu.VMEM((CH,), jnp.float32),
          pltpu.VMEM((CH,), jnp.float32),
          pltpu.VMEM((CH, WG), jnp.float32),
          pltpu.VMEM((CH, WG), jnp.float32),
          pltpu.VMEM_SHARED((NP, WG), jnp.float32),
          pltpu.SemaphoreType.DMA,
          pltpu.SemaphoreType.DMA,
          pltpu.SemaphoreType.DMA,
          pltpu.SemaphoreType.DMA,
      ],
  )
  def edge_kernel(gext, asrc, adst, srcp, dstp2, zinit, out,
                  av_s, av_d, sbufA, dbuf2A, sbufB, dbuf2B, wbufA, wbufB,
                  rows0, rows1, accS, sem_i, sem_g0, sem_g1, sem_s1):
    cid = lax.axis_index("c")
    sid = lax.axis_index("s")
    wid = cid * 16 + sid
    iota16 = lax.iota(jnp.int32, 16)
    zero16 = jnp.zeros((16,), jnp.int32)
    tile_base = wid * EPT

    def load_idx(base, sb, db2):
      row0 = pl.multiple_of(base // 128, 8)
      c1 = pltpu.async_copy(srcp.at[pl.ds(base, CH)], sb, sem_i)
      c2 = pltpu.async_copy(dstp2.at[pl.ds(row0, NSUB)], db2, sem_i)
      c1.wait()
      c2.wait()

    def fire_gathers(sb, rws, sem):
      return [
          pltpu.async_copy(gext.at[sb.at[pl.ds(j * 128, 128)]],
                           rws.at[pl.ds(j * 128, 128)], sem)
          for j in range(NSUB)
      ]

    def drain_gathers(sem, sb, rws):
      for j in range(NSUB):
        pltpu.make_async_copy(gext.at[sb.at[pl.ds(j * 128, 128)]],
                              rws.at[pl.ds(j * 128, 128)], sem).wait()

    def fire_scatters(rws, db2, sem):
      return [
          pltpu.async_copy(rws.at[pl.ds(j * 128, 128)],
                           accS.at[db2.at[j]], sem, add=True)
          for j in range(NSUB)
      ]

    def drain_scatters(sem, rws, db2):
      for j in range(NSUB):
        pltpu.make_async_copy(rws.at[pl.ds(j * 128, 128)],
                              accS.at[db2.at[j]], sem).wait()

    def compute_w(sb, db2, wb):
      for j in range(NSUB):
        def grp(gh, carry2, j=j):
          for u in range(2):
            gg = gh * 2 + u
            g = j * 8 + gg
            s16 = sb[pl.ds(g * 16, 16)]
            d16 = db2[j, pl.ds(gg * 16, 16)]
            a_s = plsc.load_gather(av_s, [s16])
            a_d = plsc.load_gather(av_d, [d16])
            e = a_s + a_d
            e = jnp.where(e >= 0.0, e, 0.2 * e)
            wb[pl.ds(g * 16, 16)] = jnp.exp(e)
          return carry2
        lax.fori_loop(0, 4, grp, 0)

    def scale_rows(rws, wb):
      def grp(g0, carry2):
        for u in range(4):
          g = g0 * 4 + u
          w = wb[pl.ds(g * 16, 16)]
          ridx = g * 16 + iota16
          for c in range(WG - 1):
            cidx = jnp.full((16,), c, jnp.int32)
            v = plsc.load_gather(rws, [ridx, cidx])
            plsc.store_scatter(rws, [ridx, cidx], v * w)
          plsc.store_scatter(rws, [ridx, jnp.full((16,), WG - 1, jnp.int32)],
                             w)
        return carry2
      lax.fori_loop(0, CH // 64, grp, 0)

    pltpu.sync_copy(asrc, av_s)
    pltpu.sync_copy(adst, av_d)

    pltpu.sync_copy(zinit.at[pl.ds(sid * RPT, RPT)],
                    accS.at[pl.ds(sid * RPT, RPT)])
    plsc.subcore_barrier()

    load_idx(tile_base, sbufA, dbuf2A)
    fire_gathers(sbufA, rows0, sem_g0)

    def pair_body(p, carry):
      baseA = pl.multiple_of(tile_base + (2 * p) * CH, CH)
      baseB = pl.multiple_of(baseA + CH, CH)

      @pl.when(p > 0)
      def _():
        drain_scatters(sem_s1, rows1, dbuf2B)

      load_idx(baseB, sbufB, dbuf2B)
      compute_w(sbufA, dbuf2A, wbufA)
      drain_gathers(sem_g0, sbufA, rows0)
      hg1 = fire_gathers(sbufB, rows1, sem_g1)
      scale_rows(rows0, wbufA)
      hs0 = fire_scatters(rows0, dbuf2A, sem_g0)
      compute_w(sbufB, dbuf2B, wbufB)
      for h in hg1:
        h.wait()
      for h in hs0:
        h.wait()
      nxt = jnp.minimum(2 * p + 2, NCHUNK - 1)
      baseN = pl.multiple_of(tile_base + nxt * CH, CH)
      load_idx(baseN, sbufA, dbuf2A)

      @pl.when(p < NPAIR - 1)
      def _():
        fire_gathers(sbufA, rows0, sem_g0)

      scale_rows(rows1, wbufB)
      fire_scatters(rows1, dbuf2B, sem_s1)
      return carry

    lax.fori_loop(0, NPAIR, pair_body, 0)
    drain_scatters(sem_s1, rows1, dbuf2B)

    plsc.subcore_barrier()
    pltpu.sync_copy(accS.at[pl.ds(sid * RPT, RPT)],
                    out.at[pl.ds(cid * NP + sid * RPT, RPT)])

  return edge_kernel


def _tc_layer1(x, W1, a1s, a1d, NP):
  N, _ = x.shape
  H = W1.shape[1]
  zrows = NP - N

  def body(x_ref, w_ref, s_ref, d_ref, g_ref, as_ref, ad_ref):
    h = jnp.dot(x_ref[...], w_ref[...], preferred_element_type=jnp.float32)
    g_ref[...] = jnp.concatenate(
        [h, jnp.ones((N, 1), jnp.float32)], axis=1)
    zp = jnp.zeros((zrows, 1), jnp.float32)
    asv = jnp.dot(h, s_ref[...], preferred_element_type=jnp.float32)
    adv = jnp.dot(h, d_ref[...], preferred_element_type=jnp.float32)
    as_ref[...] = jnp.concatenate([asv, zp], axis=0)[:, 0]
    ad_ref[...] = jnp.concatenate([adv, zp], axis=0)[:, 0]

  return pl.pallas_call(
      body,
      out_shape=[
          jax.ShapeDtypeStruct((N, H + 1), jnp.float32),
          jax.ShapeDtypeStruct((NP,), jnp.float32),
          jax.ShapeDtypeStruct((NP,), jnp.float32),
      ],
  )(x, W1, a1s.reshape(H, 1), a1d.reshape(H, 1))


def _tc_layer2(acc1, b1, W2, a2s, a2d, NP):
  H1 = W2.shape[0]
  H2 = W2.shape[1]

  def body(a_ref, b_ref, w_ref, s_ref, d_ref, g_ref, as_ref, ad_ref):
    a = a_ref[:NP] + a_ref[NP:]
    z = a[:, :H1] / (a[:, H1:] + 1e-16) + b_ref[...]
    z = jnp.maximum(z, 0.0)
    h2 = jnp.dot(z, w_ref[...], preferred_element_type=jnp.float32)
    g_ref[...] = jnp.concatenate(
        [h2, jnp.ones((NP, 1), jnp.float32)], axis=1)
    as_ref[...] = jnp.dot(
        h2, s_ref[...], preferred_element_type=jnp.float32)[:, 0]
    ad_ref[...] = jnp.dot(
        h2, d_ref[...], preferred_element_type=jnp.float32)[:, 0]

  return pl.pallas_call(
      body,
      out_shape=[
          jax.ShapeDtypeStruct((NP, H2 + 1), jnp.float32),
          jax.ShapeDtypeStruct((NP,), jnp.float32),
          jax.ShapeDtypeStruct((NP,), jnp.float32),
      ],
  )(acc1, b1.reshape(1, H1), W2, a2s.reshape(H2, 1), a2d.reshape(H2, 1))


def _tc_final(acc2, b2, Wf, bf, NP, N):
  H2 = Wf.shape[0]
  C = Wf.shape[1]

  def body(a_ref, b_ref, w_ref, bf_ref, o_ref):
    a = a_ref[:N] + a_ref[NP:NP + N]
    z = a[:, :H2] / (a[:, H2:] + 1e-16) + b_ref[...]
    o_ref[...] = jnp.dot(
        z, w_ref[...], preferred_element_type=jnp.float32) + bf_ref[...]

  return pl.pallas_call(
      body,
      out_shape=jax.ShapeDtypeStruct((N, C), jnp.float32),
  )(acc2, b2.reshape(1, H2), Wf, bf.reshape(1, C))


def kernel(x, edge_index, W1, a1_src, a1_dst, b1, W2, a2_src, a2_dst, b2,
           Wf, bf):
  N = x.shape[0]
  E = edge_index.shape[1]
  H1 = W1.shape[1]
  H2 = W2.shape[1]

  CH = 1024
  NP = ((N + 128) + 2047) // 2048 * 2048
  EP = (E + 32 * CH - 1) // (32 * CH) * (32 * CH)

  pad = EP - E
  src = edge_index[0]
  dst = edge_index[1]
  if pad:
    pidx = jnp.arange(pad, dtype=jnp.int32)
    src = jnp.concatenate([src, pidx % N])
    dst = jnp.concatenate([dst, N + (pidx % 128)])
  dst2 = dst.reshape(EP // 128, 128)

  gext1, as1, ad1 = _tc_layer1(x, W1, a1_src, a1_dst, NP)
  ek1 = _make_edge_kernel(NP, H1 + 1, EP, CH)
  acc1 = ek1(gext1, as1, ad1, src, dst2,
             jnp.zeros((NP, H1 + 1), jnp.float32))

  gext2, as2, ad2 = _tc_layer2(acc1, b1, W2, a2_src, a2_dst, NP)
  ek2 = _make_edge_kernel(NP, H2 + 1, EP, CH)
  acc2 = ek2(gext2, as2, ad2, src, dst2,
             jnp.zeros((NP, H2 + 1), jnp.float32))

  return _tc_final(acc2, b2, Wf, bf, NP, N)


# speedup vs baseline: 1.0130x; 1.0065x over previous
import functools

import jax
import jax.numpy as jnp
from jax import lax
from jax.experimental import pallas as pl
from jax.experimental.pallas import tpu as pltpu
from jax.experimental.pallas import tpu_sc as plsc


def _make_edge_kernel(NP, WG, EP, CH):
  NSUB = CH // 128
  EPT = EP // 32
  NCHUNK = EPT // CH
  NPAIR = NCHUNK // 2
  RPT = NP // 16
  assert NCHUNK % 2 == 0

  mesh = plsc.VectorSubcoreMesh(core_axis_name="c", subcore_axis_name="s")

  @functools.partial(
      pl.kernel,
      mesh=mesh,
      compiler_params=pltpu.CompilerParams(
          needs_layout_passes=False, use_tc_tiling_on_sc=False),
      out_type=jax.ShapeDtypeStruct((2 * NP, WG), jnp.float32),
      scratch_types=[
          pltpu.VMEM((NP,), jnp.float32),
          pltpu.VMEM((NP,), jnp.float32),
          pltpu.VMEM((CH,), jnp.int32),
          pltpu.VMEM((NSUB, 128), jnp.int32),
          pltpu.VMEM((CH,), jnp.int32),
          pltpu.VMEM((NSUB, 128), jnp.int32),
          pltpu.VMEM((CH,), jnp.float32),
          pltpu.VMEM((CH,), jnp.float32),
          pltpu.VMEM((CH, WG), jnp.float32),
          pltpu.VMEM((CH, WG), jnp.float32),
          pltpu.VMEM_SHARED((NP, WG), jnp.float32),
          pltpu.SemaphoreType.DMA,
          pltpu.SemaphoreType.DMA,
          pltpu.SemaphoreType.DMA,
          pltpu.SemaphoreType.DMA,
      ],
  )
  def edge_kernel(gext, asrc, adst, srcp, dstp2, zinit, out,
                  av_s, av_d, sbufA, dbuf2A, sbufB, dbuf2B, wbufA, wbufB,
                  rows0, rows1, accS, sem_i, sem_g0, sem_g1, sem_s1):
    cid = lax.axis_index("c")
    sid = lax.axis_index("s")
    wid = cid * 16 + sid
    iota16 = lax.iota(jnp.int32, 16)
    zero16 = jnp.zeros((16,), jnp.int32)
    tile_base = wid * EPT

    def load_idx(base, sb, db2):
      row0 = pl.multiple_of(base // 128, 8)
      c1 = pltpu.async_copy(srcp.at[pl.ds(base, CH)], sb, sem_i)
      c2 = pltpu.async_copy(dstp2.at[pl.ds(row0, NSUB)], db2, sem_i)
      c1.wait()
      c2.wait()

    def fire_gathers(sb, rws, sem):
      return [
          pltpu.async_copy(gext.at[sb.at[pl.ds(j * 128, 128)]],
                           rws.at[pl.ds(j * 128, 128)], sem)
          for j in range(NSUB)
      ]

    def drain_gathers(sem, sb, rws):
      for j in range(NSUB):
        pltpu.make_async_copy(gext.at[sb.at[pl.ds(j * 128, 128)]],
                              rws.at[pl.ds(j * 128, 128)], sem).wait()

    def fire_scatters(rws, db2, sem):
      return [
          pltpu.async_copy(rws.at[pl.ds(j * 128, 128)],
                           accS.at[db2.at[j]], sem, add=True)
          for j in range(NSUB)
      ]

    def drain_scatters(sem, rws, db2):
      for j in range(NSUB):
        pltpu.make_async_copy(rws.at[pl.ds(j * 128, 128)],
                              accS.at[db2.at[j]], sem).wait()

    def compute_w(sb, db2, wb):
      for j in range(NSUB):
        def grp(gh, carry2, j=j):
          for u in range(2):
            gg = gh * 2 + u
            g = j * 8 + gg
            s16 = sb[pl.ds(g * 16, 16)]
            d16 = db2[j, pl.ds(gg * 16, 16)]
            a_s = plsc.load_gather(av_s, [s16])
            a_d = plsc.load_gather(av_d, [d16])
            e = a_s + a_d
            e = jnp.where(e >= 0.0, e, 0.2 * e)
            wb[pl.ds(g * 16, 16)] = jnp.exp(e)
          return carry2
        lax.fori_loop(0, 4, grp, 0)

    def scale_rows(rws, wb):
      def grp(g0, carry2):
        for u in range(4):
          g = g0 * 4 + u
          w = wb[pl.ds(g * 16, 16)]
          ridx = g * 16 + iota16
          for c in range(WG - 1):
            cidx = jnp.full((16,), c, jnp.int32)
            v = plsc.load_gather(rws, [ridx, cidx])
            plsc.store_scatter(rws, [ridx, cidx], v * w)
          plsc.store_scatter(rws, [ridx, jnp.full((16,), WG - 1, jnp.int32)],
                             w)
        return carry2
      lax.fori_loop(0, CH // 64, grp, 0)

    pltpu.sync_copy(asrc, av_s)
    pltpu.sync_copy(adst, av_d)

    pltpu.sync_copy(zinit.at[pl.ds(sid * RPT, RPT)],
                    accS.at[pl.ds(sid * RPT, RPT)])
    plsc.subcore_barrier()

    load_idx(tile_base, sbufA, dbuf2A)
    fire_gathers(sbufA, rows0, sem_g0)

    def pair_body(p, carry):
      baseA = pl.multiple_of(tile_base + (2 * p) * CH, CH)
      baseB = pl.multiple_of(baseA + CH, CH)

      @pl.when(p > 0)
      def _():
        drain_scatters(sem_s1, rows1, dbuf2B)

      load_idx(baseB, sbufB, dbuf2B)
      compute_w(sbufA, dbuf2A, wbufA)
      drain_gathers(sem_g0, sbufA, rows0)
      hg1 = fire_gathers(sbufB, rows1, sem_g1)
      scale_rows(rows0, wbufA)
      hs0 = fire_scatters(rows0, dbuf2A, sem_g0)
      compute_w(sbufB, dbuf2B, wbufB)
      for h in hg1:
        h.wait()
      for h in hs0:
        h.wait()
      nxt = jnp.minimum(2 * p + 2, NCHUNK - 1)
      baseN = pl.multiple_of(tile_base + nxt * CH, CH)
      load_idx(baseN, sbufA, dbuf2A)

      @pl.when(p < NPAIR - 1)
      def _():
        fire_gathers(sbufA, rows0, sem_g0)

      scale_rows(rows1, wbufB)
      fire_scatters(rows1, dbuf2B, sem_s1)
      return carry

    lax.fori_loop(0, NPAIR, pair_body, 0)
    drain_scatters(sem_s1, rows1, dbuf2B)

    plsc.subcore_barrier()
    pltpu.sync_copy(accS.at[pl.ds(sid * RPT, RPT)],
                    out.at[pl.ds(cid * NP + sid * RPT, RPT)])

  return edge_kernel


def _tc_layer1(x, W1, a1s, a1d, NP):
  N, _ = x.shape
  H = W1.shape[1]
  zrows = NP - N

  def body(x_ref, w_ref, s_ref, d_ref, g_ref, as_ref, ad_ref):
    h = jnp.dot(x_ref[...], w_ref[...], preferred_element_type=jnp.float32)
    g_ref[...] = jnp.concatenate(
        [h, jnp.ones((N, 1), jnp.float32)], axis=1)
    zp = jnp.zeros((zrows, 1), jnp.float32)
    asv = jnp.dot(h, s_ref[...], preferred_element_type=jnp.float32)
    adv = jnp.dot(h, d_ref[...], preferred_element_type=jnp.float32)
    as_ref[...] = jnp.concatenate([asv, zp], axis=0)[:, 0]
    ad_ref[...] = jnp.concatenate([adv, zp], axis=0)[:, 0]

  return pl.pallas_call(
      body,
      out_shape=[
          jax.ShapeDtypeStruct((N, H + 1), jnp.float32),
          jax.ShapeDtypeStruct((NP,), jnp.float32),
          jax.ShapeDtypeStruct((NP,), jnp.float32),
      ],
  )(x, W1, a1s.reshape(H, 1), a1d.reshape(H, 1))


def _tc_layer2(acc1, b1, W2, a2s, a2d, NP):
  H1 = W2.shape[0]
  H2 = W2.shape[1]

  BN = 2048
  nb = NP // BN

  def body(a0_ref, a1_ref, b_ref, w_ref, s_ref, d_ref, g_ref, as_ref,
           ad_ref):
    a = a0_ref[...] + a1_ref[...]
    z = a[:, :H1] / (a[:, H1:] + 1e-16) + b_ref[...]
    z = jnp.maximum(z, 0.0)
    h2 = jnp.dot(z, w_ref[...], preferred_element_type=jnp.float32)
    g_ref[...] = jnp.concatenate(
        [h2, jnp.ones((BN, 1), jnp.float32)], axis=1)
    as_ref[...] = jnp.dot(
        h2, s_ref[...], preferred_element_type=jnp.float32)[:, 0]
    ad_ref[...] = jnp.dot(
        h2, d_ref[...], preferred_element_type=jnp.float32)[:, 0]

  return pl.pallas_call(
      body,
      grid=(nb,),
      in_specs=[
          pl.BlockSpec((BN, H1 + 1), lambda i: (i, 0)),
          pl.BlockSpec((BN, H1 + 1), lambda i: (i + nb, 0)),
          pl.BlockSpec((1, H1), lambda i: (0, 0)),
          pl.BlockSpec((H1, H2), lambda i: (0, 0)),
          pl.BlockSpec((H2, 1), lambda i: (0, 0)),
          pl.BlockSpec((H2, 1), lambda i: (0, 0)),
      ],
      out_specs=[
          pl.BlockSpec((BN, H2 + 1), lambda i: (i, 0)),
          pl.BlockSpec((BN,), lambda i: (i,)),
          pl.BlockSpec((BN,), lambda i: (i,)),
      ],
      out_shape=[
          jax.ShapeDtypeStruct((NP, H2 + 1), jnp.float32),
          jax.ShapeDtypeStruct((NP,), jnp.float32),
          jax.ShapeDtypeStruct((NP,), jnp.float32),
      ],
  )(acc1, acc1, b1.reshape(1, H1), W2, a2s.reshape(H2, 1),
    a2d.reshape(H2, 1))


def _tc_final(acc2, b2, Wf, bf, NP, N):
  H2 = Wf.shape[0]
  C = Wf.shape[1]

  def body(a_ref, b_ref, w_ref, bf_ref, o_ref):
    a = a_ref[:N] + a_ref[NP:NP + N]
    z = a[:, :H2] / (a[:, H2:] + 1e-16) + b_ref[...]
    o_ref[...] = jnp.dot(
        z, w_ref[...], preferred_element_type=jnp.float32) + bf_ref[...]

  return pl.pallas_call(
      body,
      out_shape=jax.ShapeDtypeStruct((N, C), jnp.float32),
  )(acc2, b2.reshape(1, H2), Wf, bf.reshape(1, C))


def kernel(x, edge_index, W1, a1_src, a1_dst, b1, W2, a2_src, a2_dst, b2,
           Wf, bf):
  N = x.shape[0]
  E = edge_index.shape[1]
  H1 = W1.shape[1]
  H2 = W2.shape[1]

  CH = 1024
  NP = ((N + 128) + 2047) // 2048 * 2048
  EP = (E + 32 * CH - 1) // (32 * CH) * (32 * CH)

  pad = EP - E
  src = edge_index[0]
  dst = edge_index[1]
  if pad:
    pidx = jnp.arange(pad, dtype=jnp.int32)
    src = jnp.concatenate([src, pidx % N])
    dst = jnp.concatenate([dst, N + (pidx % 128)])
  dst2 = dst.reshape(EP // 128, 128)

  gext1, as1, ad1 = _tc_layer1(x, W1, a1_src, a1_dst, NP)
  ek1 = _make_edge_kernel(NP, H1 + 1, EP, CH)
  acc1 = ek1(gext1, as1, ad1, src, dst2,
             jnp.zeros((NP, H1 + 1), jnp.float32))

  gext2, as2, ad2 = _tc_layer2(acc1, b1, W2, a2_src, a2_dst, NP)
  ek2 = _make_edge_kernel(NP, H2 + 1, EP, CH)
  acc2 = ek2(gext2, as2, ad2, src, dst2,
             jnp.zeros((NP, H2 + 1), jnp.float32))

  return _tc_final(acc2, b2, Wf, bf, NP, N)
